# Initial kernel scaffold; baseline (speedup 1.0000x reference)
import jax
import jax.numpy as jnp
from jax.experimental import pallas as pl


def kernel(x, pst_weight, emb_weight):
    raise NotImplementedError("write your pallas kernel here")


# speedup vs baseline: 316.3672x; 316.3672x over previous
import functools

import jax
import jax.numpy as jnp
from jax import lax
from jax.experimental import pallas as pl
from jax.experimental.pallas import tpu as pltpu
from jax.experimental.pallas import tpu_sc as plsc

B = 16384
L = 200
VOCAB = 769
NC = 2
NS = 16
NW = NC * NS
BPW = B // NW
ROWS_PER_GROUP = 16
NGROUPS = BPW // ROWS_PER_GROUP


def _pst_kernel(x_hbm, tab_hbm, out_hbm, tab_v, xb_v, out_v):
    wid = lax.axis_index("s") * NC + lax.axis_index("c")
    pltpu.sync_copy(tab_hbm, tab_v)
    pltpu.sync_copy(x_hbm.at[pl.ds(wid * (BPW * L), BPW * L)], xb_v)

    lane = lax.iota(jnp.int32, 16)
    tail_mask = lane >= 8

    def group_body(g16, _):
        outv = jnp.zeros((16,), jnp.float32)
        for g in range(ROWS_PER_GROUP):
            base = (g16 * ROWS_PER_GROUP + g) * L
            idx = xb_v[pl.ds(base, 16)]
            acc = plsc.load_gather(tab_v, [idx])
            for j in range(1, 12):
                idx = xb_v[pl.ds(base + 16 * j, 16)]
                acc = acc + plsc.load_gather(tab_v, [idx])
            idx = xb_v[pl.ds(base + L - 16, 16)]
            v = plsc.load_gather(tab_v, [idx])
            acc = acc + jnp.where(tail_mask, v, 0.0)
            rs = jnp.sum(acc)
            outv = jnp.where(lane == g, rs, outv)
        out_v[pl.ds(g16 * ROWS_PER_GROUP, 16)] = outv
        return _

    lax.fori_loop(0, NGROUPS, group_body, None)
    pltpu.sync_copy(out_v, out_hbm.at[pl.ds(wid * BPW, BPW)])


@jax.jit
def _pst_sum(x_flat, tab_flat):
    mesh = plsc.VectorSubcoreMesh(core_axis_name="c", subcore_axis_name="s")
    f = pl.kernel(
        _pst_kernel,
        out_type=jax.ShapeDtypeStruct((B,), jnp.float32),
        mesh=mesh,
        scratch_types=[
            pltpu.VMEM((VOCAB,), jnp.float32),
            pltpu.VMEM((BPW * L,), jnp.int32),
            pltpu.VMEM((BPW,), jnp.float32),
        ],
        compiler_params=pltpu.CompilerParams(needs_layout_passes=False),
    )
    return f(x_flat, tab_flat)


def kernel(x, pst_weight, emb_weight):
    x_flat = x.astype(jnp.int32).reshape(-1)
    tab_flat = pst_weight.reshape(-1)
    return _pst_sum(x_flat, tab_flat)


# speedup vs baseline: 408.8321x; 1.2923x over previous
import functools

import jax
import jax.numpy as jnp
from jax import lax
from jax.experimental import pallas as pl
from jax.experimental.pallas import tpu as pltpu
from jax.experimental.pallas import tpu_sc as plsc

B = 16384
L = 200
VOCAB = 769
NC = 2
NS = 16
NW = NC * NS
BPW = B // NW
CR = 128
NCH = BPW // CR
ROWS_PER_GROUP = 16
NGROUPS = CR // ROWS_PER_GROUP


def _row_sum(xb_v, b, r, tab_v, tail_mask):
    vals = []
    for j in range(12):
        idx = xb_v[b, r, pl.ds(16 * j, 16)]
        vals.append(plsc.load_gather(tab_v, [idx]))
    idx = xb_v[b, r, pl.ds(L - 16, 16)]
    v = plsc.load_gather(tab_v, [idx])
    vals.append(jnp.where(tail_mask, v, 0.0))
    while len(vals) > 1:
        vals = [a + c for a, c in zip(vals[::2], vals[1::2])] + (
            [vals[-1]] if len(vals) % 2 else []
        )
    return jnp.sum(vals[0])


def _pst_kernel(x_hbm, tab_hbm, out_hbm, tab_v, xb_v, out_v, sem0, sem1):
    wid = lax.axis_index("s") * NC + lax.axis_index("c")
    row0 = wid * BPW
    pltpu.sync_copy(tab_hbm, tab_v)

    sems = (sem0, sem1)
    copies = [None, None]
    for c in range(min(2, NCH)):
        copies[c] = pltpu.async_copy(
            x_hbm.at[pl.ds(row0 + c * CR, CR)], xb_v.at[c], sems[c]
        )

    lane = lax.iota(jnp.int32, 16)
    tail_mask = lane >= 8

    for c in range(NCH):
        b = c % 2
        copies[b].wait()

        def group_body(g16, _, b=b, c=c):
            outv = jnp.zeros((16,), jnp.float32)
            for g in range(ROWS_PER_GROUP):
                r = g16 * ROWS_PER_GROUP + g
                rs = _row_sum(xb_v, b, r, tab_v, tail_mask)
                outv = jnp.where(lane == g, rs, outv)
            out_v[pl.ds(c * CR + g16 * ROWS_PER_GROUP, 16)] = outv
            return _

        lax.fori_loop(0, NGROUPS, group_body, None)

        if c + 2 < NCH:
            copies[b] = pltpu.async_copy(
                x_hbm.at[pl.ds(row0 + (c + 2) * CR, CR)], xb_v.at[b], sems[b]
            )

    pltpu.sync_copy(out_v, out_hbm.at[pl.ds(row0, BPW)])


@jax.jit
def _pst_sum(x, tab_flat):
    mesh = plsc.VectorSubcoreMesh(core_axis_name="c", subcore_axis_name="s")
    f = pl.kernel(
        _pst_kernel,
        out_type=jax.ShapeDtypeStruct((B,), jnp.float32),
        mesh=mesh,
        scratch_types=[
            pltpu.VMEM((VOCAB,), jnp.float32),
            pltpu.VMEM((2, CR, L), jnp.int32),
            pltpu.VMEM((BPW,), jnp.float32),
            pltpu.SemaphoreType.DMA,
            pltpu.SemaphoreType.DMA,
        ],
        compiler_params=pltpu.CompilerParams(needs_layout_passes=False),
    )
    return f(x, tab_flat)


def kernel(x, pst_weight, emb_weight):
    return _pst_sum(x.astype(jnp.int32), pst_weight.reshape(-1))
